# Initial kernel scaffold; baseline (speedup 1.0000x reference)
import jax
import jax.numpy as jnp
from jax.experimental import pallas as pl


def kernel(x, token_table, pos_table):
    raise NotImplementedError("write your pallas kernel here")


# speedup vs baseline: 1.0531x; 1.0531x over previous
import functools

import jax
import jax.numpy as jnp
from jax import lax
from jax.experimental import pallas as pl
from jax.experimental.pallas import tpu as pltpu
from jax.experimental.pallas import tpu_sc as plsc

B = 64
S = 512
E = 128
N = B * S
NC = 2
NS = 16
NW = NC * NS
PER_W = N // NW
CHUNK = 128
NCHUNK = PER_W // CHUNK
LANES = 16
EV = E // LANES

_mesh = plsc.VectorSubcoreMesh(core_axis_name="c", subcore_axis_name="s")


@functools.partial(
    pl.kernel,
    out_type=jax.ShapeDtypeStruct((N, E), jnp.float32),
    mesh=_mesh,
    scratch_types=[
        pltpu.VMEM((NCHUNK, CHUNK), jnp.int32),
        pltpu.VMEM((S, E), jnp.float32),
        pltpu.VMEM((CHUNK, E), jnp.float32),
        pltpu.VMEM((CHUNK, E), jnp.float32),
        pltpu.SemaphoreType.DMA,
        pltpu.SemaphoreType.DMA,
        pltpu.SemaphoreType.DMA,
        pltpu.SemaphoreType.DMA,
        pltpu.SemaphoreType.DMA,
    ],
)
def _emb_lookup(x_hbm, tok_hbm, pos_hbm, out_hbm,
                idx_v, pos_v, buf0, buf1, gsem0, gsem1, ssem0, ssem1, psem):
    wid = lax.axis_index("s") * NC + lax.axis_index("c")
    base = wid * PER_W

    pos_cp = pltpu.async_copy(pos_hbm, pos_v, psem)
    pltpu.sync_copy(x_hbm.at[pl.ds(wid * NCHUNK, NCHUNK)], idx_v)

    bufs = (buf0, buf1)
    gsems = (gsem0, gsem1)
    ssems = (ssem0, ssem1)

    def start_gather(c):
        b = c % 2
        idx_row = idx_v.at[c]
        return pltpu.async_copy(tok_hbm.at[idx_row], bufs[b], gsems[b])

    gather_cp = [None, None]
    store_cp = [None, None]

    gather_cp[0] = start_gather(0)
    pos_cp.wait()

    for c in range(NCHUNK):
        b = c % 2
        if c + 1 < NCHUNK:
            bn = (c + 1) % 2
            if store_cp[bn] is not None:
                store_cp[bn].wait()
            gather_cp[bn] = start_gather(c + 1)
        gather_cp[b].wait()

        po = (c % (S // CHUNK)) * CHUNK
        buf = bufs[b]

        def add_pos(r, _):
            for e in range(EV):
                v = pos_v[po + r, pl.ds(e * LANES, LANES)]
                plsc.addupdate(buf.at[r, pl.ds(e * LANES, LANES)], v)
            return 0

        lax.fori_loop(0, CHUNK, add_pos, 0, unroll=2)

        store_cp[b] = pltpu.async_copy(
            buf, out_hbm.at[pl.ds(base + c * CHUNK, CHUNK)], ssems[b])

    store_cp[0].wait()
    store_cp[1].wait()


def kernel(x, token_table, pos_table):
    xf = x.reshape(NW * NCHUNK, CHUNK).astype(jnp.int32)
    out = _emb_lookup(xf, token_table, pos_table)
    return out.reshape(B, S, E)


# speedup vs baseline: 1.1727x; 1.1136x over previous
import functools

import jax
import jax.numpy as jnp
from jax import lax
from jax.experimental import pallas as pl
from jax.experimental.pallas import tpu as pltpu
from jax.experimental.pallas import tpu_sc as plsc

B = 64
S = 512
E = 128
N = B * S
NC = 2
NS = 16
NW = NC * NS
PER_W = N // NW
CHUNK = 128
NCHUNK = PER_W // CHUNK
LANES = 16
EV = E // LANES

_mesh = plsc.VectorSubcoreMesh(core_axis_name="c", subcore_axis_name="s")


@functools.partial(
    pl.kernel,
    out_type=jax.ShapeDtypeStruct((N, E), jnp.float32),
    mesh=_mesh,
    scratch_types=[
        pltpu.VMEM((NCHUNK, CHUNK), jnp.int32),
        pltpu.VMEM_SHARED((S, E), jnp.float32),
        pltpu.VMEM((S, E), jnp.float32),
        pltpu.VMEM((CHUNK, E), jnp.float32),
        pltpu.VMEM((CHUNK, E), jnp.float32),
        pltpu.VMEM((CHUNK, E), jnp.float32),
        pltpu.SemaphoreType.DMA,
        pltpu.SemaphoreType.DMA,
        pltpu.SemaphoreType.DMA,
        pltpu.SemaphoreType.DMA,
        pltpu.SemaphoreType.DMA,
        pltpu.SemaphoreType.DMA,
        pltpu.SemaphoreType.DMA,
    ],
)
def _emb_lookup(x_hbm, tok_hbm, pos_hbm, out_hbm,
                idx_v, pos_sh, pos_v, buf0, buf1, buf2,
                gsem0, gsem1, gsem2, ssem0, ssem1, ssem2, psem):
    sid = lax.axis_index("s")
    wid = sid * NC + lax.axis_index("c")
    base = wid * PER_W

    pltpu.sync_copy(x_hbm.at[pl.ds(wid * NCHUNK, NCHUNK)], idx_v)

    bufs = (buf0, buf1, buf2)
    gsems = (gsem0, gsem1, gsem2)
    ssems = (ssem0, ssem1, ssem2)
    NBUF = 3

    def start_gather(c):
        b = c % NBUF
        idx_row = idx_v.at[c]
        return pltpu.async_copy(tok_hbm.at[idx_row], bufs[b], gsems[b])

    gather_cp = [None] * NBUF
    store_cp = [None] * NBUF

    gather_cp[0] = start_gather(0)

    @pl.when(sid == 0)
    def _():
        pltpu.sync_copy(pos_hbm, pos_sh)

    plsc.subcore_barrier()
    pos_cp = pltpu.async_copy(pos_sh, pos_v, psem)

    gather_cp[1] = start_gather(1)
    pos_cp.wait()

    for c in range(NCHUNK):
        b = c % NBUF
        if c + 2 < NCHUNK:
            bn = (c + 2) % NBUF
            if store_cp[bn] is not None:
                store_cp[bn].wait()
            gather_cp[bn] = start_gather(c + 2)
        gather_cp[b].wait()

        po = (c % (S // CHUNK)) * CHUNK
        buf = bufs[b]

        def add_pos(r, _):
            for e in range(EV):
                v = pos_v[po + r, pl.ds(e * LANES, LANES)]
                plsc.addupdate(buf.at[r, pl.ds(e * LANES, LANES)], v)
            return 0

        lax.fori_loop(0, CHUNK, add_pos, 0, unroll=4)

        store_cp[b] = pltpu.async_copy(
            buf, out_hbm.at[pl.ds(base + c * CHUNK, CHUNK)], ssems[b])

    for b in range(NBUF):
        store_cp[b].wait()


def kernel(x, token_table, pos_table):
    xf = x.reshape(NW * NCHUNK, CHUNK).astype(jnp.int32)
    out = _emb_lookup(xf, token_table, pos_table)
    return out.reshape(B, S, E)


# speedup vs baseline: 1.2830x; 1.0940x over previous
import functools

import jax
import jax.numpy as jnp
from jax import lax
from jax.experimental import pallas as pl
from jax.experimental.pallas import tpu as pltpu
from jax.experimental.pallas import tpu_sc as plsc

B = 64
S = 512
E = 128
N = B * S
NC = 2
NS = 16
NW = NC * NS
PER_W = N // NW
CHUNK = 128
NCHUNK = PER_W // CHUNK
NBUF = 4

_mesh = plsc.VectorSubcoreMesh(core_axis_name="c", subcore_axis_name="s")


@functools.partial(
    pl.kernel,
    out_type=jax.ShapeDtypeStruct((N, E), jnp.float32),
    mesh=_mesh,
    scratch_types=[
        pltpu.VMEM((NCHUNK, CHUNK), jnp.int32),
        pltpu.VMEM_SHARED((S, E), jnp.float32),
        [pltpu.VMEM((CHUNK, E), jnp.float32) for _ in range(NBUF)],
        [pltpu.SemaphoreType.DMA for _ in range(NBUF)],
        [pltpu.SemaphoreType.DMA for _ in range(NBUF)],
        [pltpu.SemaphoreType.DMA for _ in range(NBUF)],
    ],
)
def _emb_lookup(x_hbm, tok_hbm, pos_hbm, out_hbm,
                idx_v, pos_sh, bufs, psems, gsems, ssems):
    sid = lax.axis_index("s")
    wid = sid * NC + lax.axis_index("c")
    base = wid * PER_W

    pltpu.sync_copy(x_hbm.at[pl.ds(wid * NCHUNK, NCHUNK)], idx_v)

    @pl.when(sid == 0)
    def _():
        pltpu.sync_copy(pos_hbm, pos_sh)

    plsc.subcore_barrier()

    def start_prefill(c):
        b = c % NBUF
        po = (c % (S // CHUNK)) * CHUNK
        return pltpu.async_copy(pos_sh.at[pl.ds(po, CHUNK)], bufs[b], psems[b])

    def start_gather_add(c):
        b = c % NBUF
        idx_row = idx_v.at[c]
        return pltpu.async_copy(tok_hbm.at[idx_row], bufs[b], gsems[b],
                                add=True)

    pre_cp = [None] * NBUF
    gather_cp = [None] * NBUF
    store_cp = [None] * NBUF

    pre_cp[0] = start_prefill(0)
    pre_cp[1] = start_prefill(1)
    pre_cp[0].wait()
    gather_cp[0] = start_gather_add(0)

    for c in range(NCHUNK):
        b = c % NBUF
        if c + 2 < NCHUNK:
            bn = (c + 2) % NBUF
            if store_cp[bn] is not None:
                store_cp[bn].wait()
            pre_cp[bn] = start_prefill(c + 2)
        if c + 1 < NCHUNK:
            bm = (c + 1) % NBUF
            pre_cp[bm].wait()
            gather_cp[bm] = start_gather_add(c + 1)
        gather_cp[b].wait()
        store_cp[b] = pltpu.async_copy(
            bufs[b], out_hbm.at[pl.ds(base + c * CHUNK, CHUNK)], ssems[b])

    for b in range(min(NBUF, NCHUNK)):
        store_cp[b].wait()


def kernel(x, token_table, pos_table):
    xf = x.reshape(NW * NCHUNK, CHUNK).astype(jnp.int32)
    out = _emb_lookup(xf, token_table, pos_table)
    return out.reshape(B, S, E)


# speedup vs baseline: 1.3040x; 1.0164x over previous
import functools

import jax
import jax.numpy as jnp
from jax import lax
from jax.experimental import pallas as pl
from jax.experimental.pallas import tpu as pltpu
from jax.experimental.pallas import tpu_sc as plsc

B = 64
S = 512
E = 128
N = B * S
NC = 2
NS = 16
NW = NC * NS
PER_W = N // NW
CHUNK = 128
NCHUNK = PER_W // CHUNK
NBUF = 4

_mesh = plsc.VectorSubcoreMesh(core_axis_name="c", subcore_axis_name="s")


SEQ_PER_W = PER_W // S
CH_PER_SEQ = S // CHUNK


@functools.partial(
    pl.kernel,
    out_type=jax.ShapeDtypeStruct((B, S, E), jnp.float32),
    mesh=_mesh,
    scratch_types=[
        pltpu.VMEM((SEQ_PER_W, S), jnp.int32),
        pltpu.VMEM_SHARED((S, E), jnp.float32),
        [pltpu.VMEM((CHUNK, E), jnp.float32) for _ in range(NBUF)],
        [pltpu.SemaphoreType.DMA for _ in range(NBUF)],
        [pltpu.SemaphoreType.DMA for _ in range(NBUF)],
        [pltpu.SemaphoreType.DMA for _ in range(NBUF)],
    ],
)
def _emb_lookup(x_hbm, tok_hbm, pos_hbm, out_hbm,
                idx_v, pos_sh, bufs, psems, gsems, ssems):
    sid = lax.axis_index("s")
    wid = sid * NC + lax.axis_index("c")
    b0 = wid * SEQ_PER_W

    pltpu.sync_copy(x_hbm.at[pl.ds(b0, SEQ_PER_W)], idx_v)

    @pl.when(sid == 0)
    def _():
        pltpu.sync_copy(pos_hbm, pos_sh)

    plsc.subcore_barrier()

    def start_prefill(c):
        b = c % NBUF
        po = (c % (S // CHUNK)) * CHUNK
        return pltpu.async_copy(pos_sh.at[pl.ds(po, CHUNK)], bufs[b], psems[b])

    def start_gather_add(c):
        b = c % NBUF
        idx_row = idx_v.at[c // CH_PER_SEQ,
                           pl.ds((c % CH_PER_SEQ) * CHUNK, CHUNK)]
        return pltpu.async_copy(tok_hbm.at[idx_row], bufs[b], gsems[b],
                                add=True)

    pre_cp = [None] * NBUF
    gather_cp = [None] * NBUF
    store_cp = [None] * NBUF

    pre_cp[0] = start_prefill(0)
    pre_cp[1] = start_prefill(1)
    pre_cp[0].wait()
    gather_cp[0] = start_gather_add(0)

    for c in range(NCHUNK):
        b = c % NBUF
        if c + 2 < NCHUNK:
            bn = (c + 2) % NBUF
            if store_cp[bn] is not None:
                store_cp[bn].wait()
            pre_cp[bn] = start_prefill(c + 2)
        if c + 1 < NCHUNK:
            bm = (c + 1) % NBUF
            pre_cp[bm].wait()
            gather_cp[bm] = start_gather_add(c + 1)
        gather_cp[b].wait()
        store_cp[b] = pltpu.async_copy(
            bufs[b],
            out_hbm.at[b0 + c // CH_PER_SEQ,
                       pl.ds((c % CH_PER_SEQ) * CHUNK, CHUNK)],
            ssems[b])

    for b in range(min(NBUF, NCHUNK)):
        store_cp[b].wait()


def kernel(x, token_table, pos_table):
    return _emb_lookup(x.astype(jnp.int32), token_table, pos_table)


# speedup vs baseline: 1.3334x; 1.0226x over previous
import functools

import jax
import jax.numpy as jnp
from jax import lax
from jax.experimental import pallas as pl
from jax.experimental.pallas import tpu as pltpu
from jax.experimental.pallas import tpu_sc as plsc

B = 64
S = 512
E = 128
N = B * S
NC = 2
NS = 16
NW = NC * NS
PER_W = N // NW
CHUNK = 128
NCHUNK = PER_W // CHUNK
NBUF = 6
PRE_AHEAD = 4
GAT_AHEAD = 2

_mesh = plsc.VectorSubcoreMesh(core_axis_name="c", subcore_axis_name="s")


SEQ_PER_W = PER_W // S
CH_PER_SEQ = S // CHUNK


@functools.partial(
    pl.kernel,
    out_type=jax.ShapeDtypeStruct((B, S, E), jnp.float32),
    mesh=_mesh,
    scratch_types=[
        pltpu.VMEM((SEQ_PER_W, S), jnp.int32),
        pltpu.VMEM_SHARED((S, E), jnp.float32),
        [pltpu.VMEM((CHUNK, E), jnp.float32) for _ in range(NBUF)],
        [pltpu.SemaphoreType.DMA for _ in range(NBUF)],
        [pltpu.SemaphoreType.DMA for _ in range(NBUF)],
        [pltpu.SemaphoreType.DMA for _ in range(NBUF)],
    ],
)
def _emb_lookup(x_hbm, tok_hbm, pos_hbm, out_hbm,
                idx_v, pos_sh, bufs, psems, gsems, ssems):
    sid = lax.axis_index("s")
    wid = sid * NC + lax.axis_index("c")
    b0 = wid * SEQ_PER_W

    idx_cp = pltpu.async_copy(x_hbm.at[pl.ds(b0, SEQ_PER_W)], idx_v,
                              gsems[NBUF - 1])

    @pl.when(sid == 0)
    def _():
        pltpu.sync_copy(pos_hbm, pos_sh)

    plsc.subcore_barrier()

    def start_prefill(c):
        b = c % NBUF
        po = (c % (S // CHUNK)) * CHUNK
        return pltpu.async_copy(pos_sh.at[pl.ds(po, CHUNK)], bufs[b], psems[b])

    def start_gather_add(c):
        b = c % NBUF
        idx_row = idx_v.at[c // CH_PER_SEQ,
                           pl.ds((c % CH_PER_SEQ) * CHUNK, CHUNK)]
        return pltpu.async_copy(tok_hbm.at[idx_row], bufs[b], gsems[b],
                                add=True)

    pre_cp = [None] * NBUF
    gather_cp = [None] * NBUF
    store_cp = [None] * NBUF

    for c in range(PRE_AHEAD):
        pre_cp[c % NBUF] = start_prefill(c)
    idx_cp.wait()
    for c in range(GAT_AHEAD):
        pre_cp[c % NBUF].wait()
        gather_cp[c % NBUF] = start_gather_add(c)

    for c in range(NCHUNK):
        b = c % NBUF
        if c + PRE_AHEAD < NCHUNK:
            bn = (c + PRE_AHEAD) % NBUF
            if store_cp[bn] is not None:
                store_cp[bn].wait()
            pre_cp[bn] = start_prefill(c + PRE_AHEAD)
        if c + GAT_AHEAD < NCHUNK:
            bm = (c + GAT_AHEAD) % NBUF
            pre_cp[bm].wait()
            gather_cp[bm] = start_gather_add(c + GAT_AHEAD)
        gather_cp[b].wait()
        store_cp[b] = pltpu.async_copy(
            bufs[b],
            out_hbm.at[b0 + c // CH_PER_SEQ,
                       pl.ds((c % CH_PER_SEQ) * CHUNK, CHUNK)],
            ssems[b])

    for b in range(min(NBUF, NCHUNK)):
        store_cp[b].wait()


def kernel(x, token_table, pos_table):
    return _emb_lookup(x.astype(jnp.int32), token_table, pos_table)


# speedup vs baseline: 1.3374x; 1.0030x over previous
import functools

import jax
import jax.numpy as jnp
from jax import lax
from jax.experimental import pallas as pl
from jax.experimental.pallas import tpu as pltpu
from jax.experimental.pallas import tpu_sc as plsc

B = 64
S = 512
E = 128
N = B * S
NC = 2
NS = 16
NW = NC * NS
PER_W = N // NW
CHUNK = 128
NCHUNK = PER_W // CHUNK
NBUF = 6
PRE_AHEAD = 4
GAT_AHEAD = 2

_mesh = plsc.VectorSubcoreMesh(core_axis_name="c", subcore_axis_name="s")


SEQ_PER_W = PER_W // S
CH_PER_SEQ = S // CHUNK


@functools.partial(
    pl.kernel,
    out_type=jax.ShapeDtypeStruct((B, S, E), jnp.float32),
    mesh=_mesh,
    scratch_types=[
        pltpu.VMEM((SEQ_PER_W, S), jnp.int32),
        pltpu.VMEM_SHARED((S, E), jnp.float32),
        [pltpu.VMEM((CHUNK, E), jnp.float32) for _ in range(NBUF)],
        [pltpu.SemaphoreType.DMA for _ in range(NBUF)],
        [pltpu.SemaphoreType.DMA for _ in range(NBUF)],
        [pltpu.SemaphoreType.DMA for _ in range(NBUF)],
    ],
)
def _emb_lookup(x_hbm, tok_hbm, pos_hbm, out_hbm,
                idx_v, pos_sh, bufs, psems, gsems, ssems):
    sid = lax.axis_index("s")
    wid = sid * NC + lax.axis_index("c")
    b0 = wid * SEQ_PER_W

    idx_cp = pltpu.async_copy(x_hbm.at[pl.ds(b0, SEQ_PER_W)], idx_v,
                              gsems[NBUF - 1])

    prows = S // NS
    pltpu.sync_copy(pos_hbm.at[pl.ds(sid * prows, prows)],
                    pos_sh.at[pl.ds(sid * prows, prows)])

    plsc.subcore_barrier()

    def start_prefill(c):
        b = c % NBUF
        po = (c % (S // CHUNK)) * CHUNK
        return pltpu.async_copy(pos_sh.at[pl.ds(po, CHUNK)], bufs[b], psems[b])

    def start_gather_add(c):
        b = c % NBUF
        idx_row = idx_v.at[c // CH_PER_SEQ,
                           pl.ds((c % CH_PER_SEQ) * CHUNK, CHUNK)]
        return pltpu.async_copy(tok_hbm.at[idx_row], bufs[b], gsems[b],
                                add=True)

    pre_cp = [None] * NBUF
    gather_cp = [None] * NBUF
    store_cp = [None] * NBUF

    for c in range(PRE_AHEAD):
        pre_cp[c % NBUF] = start_prefill(c)
    idx_cp.wait()
    for c in range(GAT_AHEAD):
        pre_cp[c % NBUF].wait()
        gather_cp[c % NBUF] = start_gather_add(c)

    for c in range(NCHUNK):
        b = c % NBUF
        if c + PRE_AHEAD < NCHUNK:
            bn = (c + PRE_AHEAD) % NBUF
            if store_cp[bn] is not None:
                store_cp[bn].wait()
            pre_cp[bn] = start_prefill(c + PRE_AHEAD)
        if c + GAT_AHEAD < NCHUNK:
            bm = (c + GAT_AHEAD) % NBUF
            pre_cp[bm].wait()
            gather_cp[bm] = start_gather_add(c + GAT_AHEAD)
        gather_cp[b].wait()
        store_cp[b] = pltpu.async_copy(
            bufs[b],
            out_hbm.at[b0 + c // CH_PER_SEQ,
                       pl.ds((c % CH_PER_SEQ) * CHUNK, CHUNK)],
            ssems[b])

    for b in range(min(NBUF, NCHUNK)):
        store_cp[b].wait()


def kernel(x, token_table, pos_table):
    return _emb_lookup(x.astype(jnp.int32), token_table, pos_table)


# speedup vs baseline: 1.3503x; 1.0097x over previous
import functools

import jax
import jax.numpy as jnp
from jax import lax
from jax.experimental import pallas as pl
from jax.experimental.pallas import tpu as pltpu
from jax.experimental.pallas import tpu_sc as plsc

B = 64
S = 512
E = 128
N = B * S
NC = 2
NS = 16
NW = NC * NS
PER_W = N // NW
CHUNK = 128
NCHUNK = PER_W // CHUNK
NBUF = 7
PRE_AHEAD = 5
GAT_AHEAD = 3

_mesh = plsc.VectorSubcoreMesh(core_axis_name="c", subcore_axis_name="s")


SEQ_PER_W = PER_W // S
CH_PER_SEQ = S // CHUNK


@functools.partial(
    pl.kernel,
    out_type=jax.ShapeDtypeStruct((B, S, E), jnp.float32),
    mesh=_mesh,
    scratch_types=[
        pltpu.VMEM((SEQ_PER_W, S), jnp.int32),
        pltpu.VMEM_SHARED((S, E), jnp.float32),
        [pltpu.VMEM((CHUNK, E), jnp.float32) for _ in range(NBUF)],
        [pltpu.SemaphoreType.DMA for _ in range(NBUF)],
        [pltpu.SemaphoreType.DMA for _ in range(NBUF)],
        [pltpu.SemaphoreType.DMA for _ in range(NBUF)],
    ],
)
def _emb_lookup(x_hbm, tok_hbm, pos_hbm, out_hbm,
                idx_v, pos_sh, bufs, psems, gsems, ssems):
    sid = lax.axis_index("s")
    wid = sid * NC + lax.axis_index("c")
    b0 = wid * SEQ_PER_W

    idx_cp = pltpu.async_copy(x_hbm.at[pl.ds(b0, SEQ_PER_W)], idx_v,
                              gsems[NBUF - 1])

    prows = S // NS
    pltpu.sync_copy(pos_hbm.at[pl.ds(sid * prows, prows)],
                    pos_sh.at[pl.ds(sid * prows, prows)])

    plsc.subcore_barrier()

    def start_prefill(c):
        b = c % NBUF
        po = (c % (S // CHUNK)) * CHUNK
        return pltpu.async_copy(pos_sh.at[pl.ds(po, CHUNK)], bufs[b], psems[b])

    def start_gather_add(c):
        b = c % NBUF
        idx_row = idx_v.at[c // CH_PER_SEQ,
                           pl.ds((c % CH_PER_SEQ) * CHUNK, CHUNK)]
        return pltpu.async_copy(tok_hbm.at[idx_row], bufs[b], gsems[b],
                                add=True)

    pre_cp = [None] * NBUF
    gather_cp = [None] * NBUF
    store_cp = [None] * NBUF

    for c in range(PRE_AHEAD):
        pre_cp[c % NBUF] = start_prefill(c)
    idx_cp.wait()
    for c in range(GAT_AHEAD):
        pre_cp[c % NBUF].wait()
        gather_cp[c % NBUF] = start_gather_add(c)

    for c in range(NCHUNK):
        b = c % NBUF
        if c + PRE_AHEAD < NCHUNK:
            bn = (c + PRE_AHEAD) % NBUF
            if store_cp[bn] is not None:
                store_cp[bn].wait()
            pre_cp[bn] = start_prefill(c + PRE_AHEAD)
        if c + GAT_AHEAD < NCHUNK:
            bm = (c + GAT_AHEAD) % NBUF
            pre_cp[bm].wait()
            gather_cp[bm] = start_gather_add(c + GAT_AHEAD)
        gather_cp[b].wait()
        store_cp[b] = pltpu.async_copy(
            bufs[b],
            out_hbm.at[b0 + c // CH_PER_SEQ,
                       pl.ds((c % CH_PER_SEQ) * CHUNK, CHUNK)],
            ssems[b])

    for b in range(min(NBUF, NCHUNK)):
        store_cp[b].wait()


def kernel(x, token_table, pos_table):
    return _emb_lookup(x.astype(jnp.int32), token_table, pos_table)


# speedup vs baseline: 1.3615x; 1.0083x over previous
import functools

import jax
import jax.numpy as jnp
from jax import lax
from jax.experimental import pallas as pl
from jax.experimental.pallas import tpu as pltpu
from jax.experimental.pallas import tpu_sc as plsc

B = 64
S = 512
E = 128
N = B * S
NC = 2
NS = 16
NW = NC * NS
PER_W = N // NW
CHUNK = 128
NCHUNK = PER_W // CHUNK
NBUF = 7
PRE_AHEAD = 5
GAT_AHEAD = 3

_mesh = plsc.VectorSubcoreMesh(core_axis_name="c", subcore_axis_name="s")


SEQ_PER_W = PER_W // S
CH_PER_SEQ = S // CHUNK


@functools.partial(
    pl.kernel,
    out_type=jax.ShapeDtypeStruct((B, S, E), jnp.float32),
    mesh=_mesh,
    scratch_types=[
        pltpu.VMEM((SEQ_PER_W, S), jnp.int32),
        pltpu.VMEM_SHARED((S, E), jnp.float32),
        [pltpu.VMEM((CHUNK, E), jnp.float32) for _ in range(NBUF)],
        [pltpu.SemaphoreType.DMA for _ in range(NBUF)],
        [pltpu.SemaphoreType.DMA for _ in range(NBUF)],
        [pltpu.SemaphoreType.DMA for _ in range(NBUF)],
    ],
)
def _emb_lookup(x_hbm, tok_hbm, pos_hbm, out_hbm,
                idx_v, pos_sh, bufs, psems, gsems, ssems):
    sid = lax.axis_index("s")
    wid = sid * NC + lax.axis_index("c")
    b0 = wid * SEQ_PER_W

    idx_cp = pltpu.async_copy(x_hbm.at[pl.ds(b0, SEQ_PER_W)], idx_v,
                              gsems[NBUF - 1])

    prows = S // NS
    pltpu.sync_copy(pos_hbm.at[pl.ds(sid * prows, prows)],
                    pos_sh.at[pl.ds(sid * prows, prows)])

    plsc.subcore_barrier()

    rot = lax.rem(wid, CH_PER_SEQ)

    def chunk_coords(k):
        seq = k // CH_PER_SEQ
        po = lax.rem(jnp.int32(k) + rot, CH_PER_SEQ) * CHUNK
        return seq, po

    def start_prefill(k):
        b = k % NBUF
        seq, po = chunk_coords(k)
        return pltpu.async_copy(pos_sh.at[pl.ds(po, CHUNK)], bufs[b], psems[b])

    def start_gather_add(k):
        b = k % NBUF
        seq, po = chunk_coords(k)
        idx_row = idx_v.at[seq, pl.ds(po, CHUNK)]
        return pltpu.async_copy(tok_hbm.at[idx_row], bufs[b], gsems[b],
                                add=True)

    pre_cp = [None] * NBUF
    gather_cp = [None] * NBUF
    store_cp = [None] * NBUF

    for c in range(PRE_AHEAD):
        pre_cp[c % NBUF] = start_prefill(c)
    idx_cp.wait()
    for c in range(GAT_AHEAD):
        pre_cp[c % NBUF].wait()
        gather_cp[c % NBUF] = start_gather_add(c)

    for c in range(NCHUNK):
        b = c % NBUF
        if c + PRE_AHEAD < NCHUNK:
            bn = (c + PRE_AHEAD) % NBUF
            if store_cp[bn] is not None:
                store_cp[bn].wait()
            pre_cp[bn] = start_prefill(c + PRE_AHEAD)
        if c + GAT_AHEAD < NCHUNK:
            bm = (c + GAT_AHEAD) % NBUF
            pre_cp[bm].wait()
            gather_cp[bm] = start_gather_add(c + GAT_AHEAD)
        gather_cp[b].wait()
        seq, po = chunk_coords(c)
        store_cp[b] = pltpu.async_copy(
            bufs[b], out_hbm.at[b0 + seq, pl.ds(po, CHUNK)], ssems[b])

    for b in range(min(NBUF, NCHUNK)):
        store_cp[b].wait()


def kernel(x, token_table, pos_table):
    return _emb_lookup(x.astype(jnp.int32), token_table, pos_table)
